# baseline (speedup 1.0000x reference)
import jax
import jax.numpy as jnp
import numpy as np
from jax.experimental import pallas as pl

_B = 16
_N = 1024
_D = 1152
_K = 16
_OUT = 32
_BK = _B * _K
_BO = _B * _OUT


def _silu(v):
    return v * jax.nn.sigmoid(v)


def _cond_kernel(t_ref, fr_ref, wt1_ref, bt1_ref, wt2_ref, bt2_ref,
                 y_ref, ytab_ref, s_ref):
    args = t_ref[...] * fr_ref[...]
    emb = jnp.concatenate([jnp.sin(args), jnp.cos(args)], axis=-1)
    h = jnp.dot(emb, wt1_ref[...], preferred_element_type=jnp.float32)
    h = _silu(h + bt1_ref[...])
    temb = jnp.dot(h, wt2_ref[...], preferred_element_type=jnp.float32)
    temb = temb + bt2_ref[...]
    n_cls = ytab_ref.shape[0]
    iota = jax.lax.broadcasted_iota(jnp.int32, (_B, n_cls), 1)
    onehot = (iota == y_ref[...]).astype(jnp.float32)
    yemb = jnp.dot(onehot, ytab_ref[...], preferred_element_type=jnp.float32)
    s_ref[...] = _silu(temb + yemb)


def _gather_kernel(x_ref, out_ref):
    xs = jnp.transpose(x_ref[...], (1, 0, 2))
    xs = xs.reshape(8 * 64, 64)
    xtr = jnp.transpose(xs)
    for ii in range(4):
        v = xtr[:, ii * 128:(ii + 1) * 128]
        v2 = v.reshape(32, 2, 128)
        a = jnp.concatenate([v2[:, 0, :], v2[:, 1, :]], axis=1)
        out_ref[ii * 32:(ii + 1) * 32, :] = a.astype(jnp.bfloat16)


def _esel():
    kk = jax.lax.broadcasted_iota(jnp.int32, (_K, _BK), 0)
    ll = jax.lax.broadcasted_iota(jnp.int32, (_K, _BK), 1)
    kl = (ll % 4) * 4 + ((ll // 64) % 2) * 2 + ll // 128
    return (kk == kl).astype(jnp.float32)


def _pre_kernel(s_ref, wada_ref, bada_ref, pos_ref, bp_ref, wpt_ref,
                wproj_ref, bproj_ref,
                qw_ref, pb_ref, pn_ref, gbig_ref, sbig_ref, wbarbig_ref,
                mbig_ref, posw_ref, soff_ref):
    ada = jnp.dot(s_ref[...], wada_ref[...], preferred_element_type=jnp.float32)
    ada = ada + bada_ref[...]
    shift = ada[:, :_D]
    sc1 = 1.0 + ada[:, _D:]

    posq = pos_ref[...] + bp_ref[...]
    pb_ref[...] = jnp.mean(posq, axis=1, keepdims=True)
    pn_ref[...] = jnp.sum(posq * posq, axis=1, keepdims=True)
    qw_ref[...] = jnp.dot(posq, wpt_ref[...], preferred_element_type=jnp.float32)

    wpt = wpt_ref[...]
    g = jax.lax.dot_general(wpt, wpt, (((0,), (0,)), ((), ())),
                            preferred_element_type=jnp.float32)
    wbarc = jax.lax.dot_general(
        wpt, jnp.full((_D, 1), 1.0 / _D, jnp.float32),
        (((0,), (0,)), ((), ())), preferred_element_type=jnp.float32)

    bi = jax.lax.broadcasted_iota(jnp.int32, (_B, _BO), 0)
    ci = jax.lax.broadcasted_iota(jnp.int32, (_B, _BO), 1)
    rsel = (bi == ci // _OUT).astype(jnp.float32)
    oi = jax.lax.broadcasted_iota(jnp.int32, (_OUT, _BO), 0)
    cj = jax.lax.broadcasted_iota(jnp.int32, (_OUT, _BO), 1)
    tsel = (oi == cj % _OUT).astype(jnp.float32)
    scale_exp = jax.lax.dot_general(sc1, rsel, (((0,), (0,)), ((), ())),
                                    preferred_element_type=jnp.float32)
    wtile = jnp.dot(wproj_ref[...], tsel, preferred_element_type=jnp.float32)
    wball = (scale_exp * wtile).astype(jnp.bfloat16)

    posw_ref[...] = jnp.dot(posq.astype(jnp.bfloat16), wball,
                            preferred_element_type=jnp.float32)
    mball = jax.lax.dot_general(
        wpt.astype(jnp.bfloat16), wball, (((0,), (0,)), ((), ())),
        preferred_element_type=jnp.float32)

    esel = _esel()
    gex = jax.lax.dot_general(esel, g, (((0,), (0,)), ((), ())),
                              preferred_element_type=jnp.float32)
    rg = jax.lax.broadcasted_iota(jnp.int32, (_BK, _BK), 0)
    cg = jax.lax.broadcasted_iota(jnp.int32, (_BK, _BK), 1)
    bmask = ((rg // 4) % _B == (cg // 4) % _B).astype(jnp.float32)
    gbig_ref[...] = jnp.dot(gex, esel,
                            preferred_element_type=jnp.float32) * bmask
    rs_ = jax.lax.broadcasted_iota(jnp.int32, (_BK, _B), 0)
    cs_ = jax.lax.broadcasted_iota(jnp.int32, (_BK, _B), 1)
    dsel = ((rs_ // 4) % _B == cs_).astype(jnp.float32)
    sbig_ref[...] = dsel
    wbar_r = jax.lax.dot_general(esel, wbarc, (((0,), (0,)), ((), ())),
                                 preferred_element_type=jnp.float32)
    wbarbig_ref[...] = dsel * wbar_r
    rm = jax.lax.broadcasted_iota(jnp.int32, (_BK, _BO), 0)
    cm = jax.lax.broadcasted_iota(jnp.int32, (_BK, _BO), 1)
    mex = jax.lax.dot_general(esel, mball, (((0,), (0,)), ((), ())),
                              preferred_element_type=jnp.float32)
    mbig_ref[...] = (mex * ((rm // 4) % _B == cm // _OUT
                            ).astype(jnp.float32)).astype(jnp.bfloat16)

    s_all = jnp.dot(sc1, wproj_ref[...], preferred_element_type=jnp.float32)
    off_all = jnp.dot(shift, wproj_ref[...],
                      preferred_element_type=jnp.float32) + bproj_ref[...]
    soff_ref[...] = jnp.concatenate(
        [s_all.reshape(1, _B, _OUT), off_all.reshape(1, _B, _OUT)], axis=0)


def _main_kernel(xt_ref, gbig_ref, sbig_ref, wbarbig_ref, qw_ref, pb_ref,
                 pn_ref, mbig_ref, posw_ref, soff_ref, out_ref):
    a2 = xt_ref[...]
    af2 = a2.astype(jnp.float32)
    p1g = jnp.dot(af2, gbig_ref[...], preferred_element_type=jnp.float32)
    qwt = jnp.dot(qw_ref[...], _esel(),
                  preferred_element_type=jnp.float32)
    v2 = af2 * (p1g + 2.0 * qwt)
    gqc = jnp.dot(v2, sbig_ref[...], preferred_element_type=jnp.float32)
    mu2 = jnp.dot(af2, wbarbig_ref[...],
                  preferred_element_type=jnp.float32) + pb_ref[...]
    msq = (gqc + pn_ref[...]) * (1.0 / _D)
    rs2 = jax.lax.rsqrt(msq - mu2 * mu2 + 1e-6)
    raw = jnp.dot(a2, mbig_ref[...],
                  preferred_element_type=jnp.float32) + posw_ref[...]
    ri = jax.lax.broadcasted_iota(jnp.int32, (_B, _BO), 0)
    ci = jax.lax.broadcasted_iota(jnp.int32, (_B, _BO), 1)
    rb = (ri == ci // _OUT).astype(jnp.float32)
    rsb = jnp.dot(rs2, rb, preferred_element_type=jnp.float32)
    rsmub = jnp.dot(rs2 * mu2, rb, preferred_element_type=jnp.float32)
    out_ref[...] = (rsb * raw - rsmub * soff_ref[0] + soff_ref[1])


def kernel(x, t, y, W_patch, b_patch, pos_embed, freqs, W_t1, b_t1, W_t2, b_t2,
           y_table, W_ada, b_ada, W_proj, b_proj):
    x4 = x.reshape(_B * 4, 64, 64)
    xt2 = pl.pallas_call(
        _gather_kernel,
        grid=(8,),
        in_specs=[pl.BlockSpec((_B * 4, 8, 64), lambda s: (0, s, 0))],
        out_specs=pl.BlockSpec((128, _BK), lambda s: (s, 0)),
        out_shape=jax.ShapeDtypeStruct((_N, _BK), jnp.bfloat16),
    )(x4)

    t2 = t.reshape(_B, 1)
    fr2 = freqs.reshape(1, _D // 2)
    y2 = y.reshape(_B, 1).astype(jnp.int32)
    pos2 = pos_embed.reshape(_N, _D)
    wpt = W_patch.T

    s = pl.pallas_call(
        _cond_kernel,
        out_shape=jax.ShapeDtypeStruct((_B, _D), jnp.float32),
    )(t2, fr2, W_t1, b_t1.reshape(1, _D), W_t2, b_t2.reshape(1, _D),
      y2, y_table)

    (qw, pb, pn, gbig, sbig, wbarbig, mbig, posw, soff_u) = pl.pallas_call(
        _pre_kernel,
        out_shape=(jax.ShapeDtypeStruct((_N, _K), jnp.float32),
                   jax.ShapeDtypeStruct((_N, 1), jnp.float32),
                   jax.ShapeDtypeStruct((_N, 1), jnp.float32),
                   jax.ShapeDtypeStruct((_BK, _BK), jnp.float32),
                   jax.ShapeDtypeStruct((_BK, _B), jnp.float32),
                   jax.ShapeDtypeStruct((_BK, _B), jnp.float32),
                   jax.ShapeDtypeStruct((_BK, _BO), jnp.bfloat16),
                   jax.ShapeDtypeStruct((_N, _BO), jnp.float32),
                   jax.ShapeDtypeStruct((2, _B, _OUT), jnp.float32)),
    )(s, W_ada, b_ada.reshape(1, 2 * _D), pos2, b_patch.reshape(1, _D), wpt,
      W_proj, b_proj.reshape(1, _OUT))
    soff = soff_u.reshape(2, 1, _BO)

    out_all = pl.pallas_call(
        _main_kernel,
        out_shape=jax.ShapeDtypeStruct((_N, _BO), jnp.float32),
    )(xt2, gbig, sbig, wbarbig, qw, pb, pn, mbig, posw, soff)
    out = out_all.reshape(_N, _B, _OUT).transpose(1, 0, 2)
    return out


# speedup vs baseline: 1.1824x; 1.1824x over previous
import jax
import jax.numpy as jnp
from jax.experimental import pallas as pl

_B = 16
_N = 1024
_D = 1152
_K = 16
_OUT = 32
_BK = _B * _K
_BO = _B * _OUT


def _silu(v):
    return v * jax.nn.sigmoid(v)


def _esel():
    kk = jax.lax.broadcasted_iota(jnp.int32, (_K, _BK), 0)
    ll = jax.lax.broadcasted_iota(jnp.int32, (_K, _BK), 1)
    kl = (ll % 4) * 4 + ((ll // 64) % 2) * 2 + ll // 128
    return (kk == kl).astype(jnp.float32)


def _cond_pre_kernel(t_ref, fr_ref, wt1_ref, bt1_ref, wt2_ref, bt2_ref,
                     y_ref, ytab_ref, wada_ref, bada_ref, pos_ref, bp_ref,
                     wp_ref, wproj_ref, bproj_ref,
                     qw_ref, pb_ref, pn_ref, gbig_ref, sbig_ref, wbarbig_ref,
                     mbig_ref, posw_ref, soff_ref):
    args = t_ref[...] * fr_ref[...]
    emb = jnp.concatenate([jnp.sin(args), jnp.cos(args)], axis=-1)
    h = jnp.dot(emb, wt1_ref[...], preferred_element_type=jnp.float32)
    h = _silu(h + bt1_ref[...])
    temb = jnp.dot(h, wt2_ref[...], preferred_element_type=jnp.float32)
    temb = temb + bt2_ref[...]
    n_cls = ytab_ref.shape[0]
    iota = jax.lax.broadcasted_iota(jnp.int32, (_B, n_cls), 1)
    onehot = (iota == y_ref[...]).astype(jnp.float32)
    yemb = jnp.dot(onehot, ytab_ref[...], preferred_element_type=jnp.float32)
    s = _silu(temb + yemb)

    ada = jnp.dot(s, wada_ref[...], preferred_element_type=jnp.float32)
    ada = ada + bada_ref[...]
    shift = ada[:, :_D]
    sc1 = 1.0 + ada[:, _D:]

    posq = pos_ref[...] + bp_ref[...]
    pb_ref[...] = jnp.mean(posq, axis=1, keepdims=True)
    pn_ref[...] = jnp.sum(posq * posq, axis=1, keepdims=True)
    wp = wp_ref[...]
    qw = jax.lax.dot_general(posq, wp, (((1,), (1,)), ((), ())),
                             preferred_element_type=jnp.float32)
    qw_ref[...] = qw
    g = jax.lax.dot_general(wp, wp, (((1,), (1,)), ((), ())),
                            preferred_element_type=jnp.float32)
    wbarc = jnp.mean(wp, axis=1, keepdims=True)

    bi = jax.lax.broadcasted_iota(jnp.int32, (_B, _BO), 0)
    ci = jax.lax.broadcasted_iota(jnp.int32, (_B, _BO), 1)
    rsel = (bi == ci // _OUT).astype(jnp.float32)
    oi = jax.lax.broadcasted_iota(jnp.int32, (_OUT, _BO), 0)
    cj = jax.lax.broadcasted_iota(jnp.int32, (_OUT, _BO), 1)
    tsel = (oi == cj % _OUT).astype(jnp.float32)
    scale_exp = jax.lax.dot_general(sc1, rsel, (((0,), (0,)), ((), ())),
                                    preferred_element_type=jnp.float32)
    wtile = jnp.dot(wproj_ref[...], tsel, preferred_element_type=jnp.float32)
    wball = (scale_exp * wtile).astype(jnp.bfloat16)

    posw_ref[...] = jnp.dot(posq.astype(jnp.bfloat16), wball,
                            preferred_element_type=jnp.float32)
    mball = jnp.dot(wp.astype(jnp.bfloat16), wball,
                    preferred_element_type=jnp.float32)

    esel = _esel()
    gex = jax.lax.dot_general(esel, g, (((0,), (0,)), ((), ())),
                              preferred_element_type=jnp.float32)
    rg = jax.lax.broadcasted_iota(jnp.int32, (_BK, _BK), 0)
    cg = jax.lax.broadcasted_iota(jnp.int32, (_BK, _BK), 1)
    bmask = ((rg // 4) % _B == (cg // 4) % _B).astype(jnp.float32)
    gbig_ref[...] = jnp.dot(gex, esel,
                            preferred_element_type=jnp.float32) * bmask
    rs_ = jax.lax.broadcasted_iota(jnp.int32, (_BK, _B), 0)
    cs_ = jax.lax.broadcasted_iota(jnp.int32, (_BK, _B), 1)
    dsel = ((rs_ // 4) % _B == cs_).astype(jnp.float32)
    sbig_ref[...] = dsel
    wbar_r = jax.lax.dot_general(esel, wbarc, (((0,), (0,)), ((), ())),
                                 preferred_element_type=jnp.float32)
    wbarbig_ref[...] = dsel * wbar_r
    rm = jax.lax.broadcasted_iota(jnp.int32, (_BK, _BO), 0)
    cm = jax.lax.broadcasted_iota(jnp.int32, (_BK, _BO), 1)
    mex = jax.lax.dot_general(esel, mball, (((0,), (0,)), ((), ())),
                              preferred_element_type=jnp.float32)
    mbig_ref[...] = (mex * ((rm // 4) % _B == cm // _OUT
                            ).astype(jnp.float32)).astype(jnp.bfloat16)

    s_all = jnp.dot(sc1, wproj_ref[...], preferred_element_type=jnp.float32)
    off_all = jnp.dot(shift, wproj_ref[...],
                      preferred_element_type=jnp.float32) + bproj_ref[...]
    soff_ref[...] = jnp.concatenate(
        [s_all.reshape(1, _B, _OUT), off_all.reshape(1, _B, _OUT)], axis=0)


def _main_kernel(x_ref, gbig_ref, sbig_ref, wbarbig_ref, qw_ref, pb_ref,
                 pn_ref, mbig_ref, posw_ref, soff_ref, out_ref):
    blocks = []
    for s in range(8):
        xs = jnp.transpose(x_ref[:, 8 * s:8 * s + 8, :], (1, 0, 2))
        xs = xs.reshape(8 * 64, 64)
        xtr = jnp.transpose(xs)
        for ii in range(4):
            v = xtr[:, ii * 128:(ii + 1) * 128]
            v2 = v.reshape(32, 2, 128)
            blocks.append(jnp.concatenate([v2[:, 0, :], v2[:, 1, :]], axis=1))
    af2 = jnp.concatenate(blocks, axis=0)
    a2 = af2.astype(jnp.bfloat16)

    p1g = jnp.dot(af2, gbig_ref[...], preferred_element_type=jnp.float32)
    qwt = jnp.dot(qw_ref[...], _esel(),
                  preferred_element_type=jnp.float32)
    v2 = af2 * (p1g + 2.0 * qwt)
    gqc = jnp.dot(v2, sbig_ref[...], preferred_element_type=jnp.float32)
    mu2 = jnp.dot(af2, wbarbig_ref[...],
                  preferred_element_type=jnp.float32) + pb_ref[...]
    msq = (gqc + pn_ref[...]) * (1.0 / _D)
    rs2 = jax.lax.rsqrt(msq - mu2 * mu2 + 1e-6)
    raw = jnp.dot(a2, mbig_ref[...],
                  preferred_element_type=jnp.float32) + posw_ref[...]
    ri = jax.lax.broadcasted_iota(jnp.int32, (_B, _BO), 0)
    ci = jax.lax.broadcasted_iota(jnp.int32, (_B, _BO), 1)
    rb = (ri == ci // _OUT).astype(jnp.float32)
    rsb = jnp.dot(rs2, rb, preferred_element_type=jnp.float32)
    rsmub = jnp.dot(rs2 * mu2, rb, preferred_element_type=jnp.float32)
    out_ref[...] = (rsb * raw - rsmub * soff_ref[0] + soff_ref[1])


def kernel(x, t, y, W_patch, b_patch, pos_embed, freqs, W_t1, b_t1, W_t2, b_t2,
           y_table, W_ada, b_ada, W_proj, b_proj):
    x4 = x.reshape(_B * 4, 64, 64)
    t2 = t.reshape(_B, 1)
    fr2 = freqs.reshape(1, _D // 2)
    y2 = y.reshape(_B, 1).astype(jnp.int32)
    pos2 = pos_embed.reshape(_N, _D)

    (qw, pb, pn, gbig, sbig, wbarbig, mbig, posw, soff_u) = pl.pallas_call(
        _cond_pre_kernel,
        out_shape=(jax.ShapeDtypeStruct((_N, _K), jnp.float32),
                   jax.ShapeDtypeStruct((_N, 1), jnp.float32),
                   jax.ShapeDtypeStruct((_N, 1), jnp.float32),
                   jax.ShapeDtypeStruct((_BK, _BK), jnp.float32),
                   jax.ShapeDtypeStruct((_BK, _B), jnp.float32),
                   jax.ShapeDtypeStruct((_BK, _B), jnp.float32),
                   jax.ShapeDtypeStruct((_BK, _BO), jnp.bfloat16),
                   jax.ShapeDtypeStruct((_N, _BO), jnp.float32),
                   jax.ShapeDtypeStruct((2, _B, _OUT), jnp.float32)),
    )(t2, fr2, W_t1, b_t1.reshape(1, _D), W_t2, b_t2.reshape(1, _D),
      y2, y_table, W_ada, b_ada.reshape(1, 2 * _D), pos2,
      b_patch.reshape(1, _D), W_patch, W_proj, b_proj.reshape(1, _OUT))
    soff = soff_u.reshape(2, 1, _BO)

    out_all = pl.pallas_call(
        _main_kernel,
        out_shape=jax.ShapeDtypeStruct((_N, _BO), jnp.float32),
    )(x4, gbig, sbig, wbarbig, qw, pb, pn, mbig, posw, soff)
    out = out_all.reshape(_N, _B, _OUT).transpose(1, 0, 2)
    return out
